# baseline (speedup 1.0000x reference)
import functools

import jax
import jax.numpy as jnp
from jax import lax
from jax.experimental import pallas as pl
from jax.experimental.pallas import tpu as pltpu
from jax.experimental.pallas import tpu_sc as plsc

_INFO = plsc.get_sparse_core_info()
_NC, _NS = _INFO.num_cores, _INFO.num_subcores
_NW = _NC * _NS

_NBUF = 4


@functools.partial(jax.jit, static_argnums=(2, 3))
def _sc_gather(table, idx, bpw, seq):
    dim = table.shape[1]
    mesh = plsc.VectorSubcoreMesh(core_axis_name="c", subcore_axis_name="s")

    scratch = (
        [pltpu.VMEM((bpw * seq,), jnp.int32)]
        + [pltpu.VMEM((bpw,), jnp.int32) for _ in range(_NBUF)]
        + [pltpu.VMEM((bpw, dim), jnp.float32) for _ in range(_NBUF)]
        + [pltpu.VMEM((dim, bpw), jnp.float32) for _ in range(_NBUF)]
        + [pltpu.SemaphoreType.DMA for _ in range(2 * _NBUF)]
    )

    @functools.partial(
        pl.kernel,
        mesh=mesh,
        out_type=jax.ShapeDtypeStruct((seq * 8 * _NW, 8, bpw), jnp.float32),
        scratch_types=scratch,
        compiler_params=pltpu.CompilerParams(
            use_tc_tiling_on_sc=False, needs_layout_passes=False
        ),
    )
    def k(table_hbm, idx_hbm, out_hbm, idx_all, *bufs):
        idxt = bufs[:_NBUF]
        rows = bufs[_NBUF : 2 * _NBUF]
        tile = bufs[2 * _NBUF : 3 * _NBUF]
        gsem = bufs[3 * _NBUF : 3 * _NBUF + _NBUF]
        ssem = bufs[3 * _NBUF + _NBUF :]

        wid = lax.axis_index("s") * _NC + lax.axis_index("c")
        lane = lax.iota(jnp.int32, 16)

        pltpu.sync_copy(idx_hbm.at[pl.ds(wid * bpw * seq, bpw * seq)], idx_all)

        def fill(s, b):
            def bv(j, carry):
                pos = (j * 16 + lane) * seq + s
                idxt[b][pl.ds(j * 16, 16)] = plsc.load_gather(idx_all, [pos])
                return carry

            lax.fori_loop(0, bpw // 16, bv, 0, unroll=True)
            pltpu.async_copy(table_hbm.at[idxt[b]], rows[b], gsem[b])

        def drain(s, b):
            pltpu.make_async_copy(table_hbm.at[idxt[b]], rows[b], gsem[b]).wait()

            def tj(j, carry):
                rvec = j * 16 + lane
                for d in range(dim):
                    cvec = jnp.full((16,), d, dtype=jnp.int32)
                    v = plsc.load_gather(rows[b], [rvec, cvec])
                    tile[b][d, pl.ds(j * 16, 16)] = v
                return carry

            lax.fori_loop(0, bpw // 16, tj, 0)
            for dq in range(8):
                pltpu.async_copy(
                    tile[b].at[pl.ds(dq * 8, 8), :],
                    out_hbm.at[(s * 8 + dq) * _NW + wid],
                    ssem[b],
                )

        def store_wait(s, b):
            for dq in range(8):
                pltpu.make_async_copy(
                    tile[b].at[pl.ds(dq * 8, 8), :],
                    out_hbm.at[(s * 8 + dq) * _NW + wid],
                    ssem[b],
                ).wait()

        fill(0, 0)
        for g in range(1, _NBUF):
            if g >= _NBUF - 1:
                store_wait(g - (_NBUF - 1), (g + 1) % _NBUF)
            fill(g, g)
            drain(g - 1, g - 1)

        def round_body(r, carry):
            for b in range(_NBUF):
                g = r * _NBUF + b
                store_wait(g - (_NBUF - 1), (b + 1) % _NBUF)
                fill(g, b)
                drain(g - 1, (b + _NBUF - 1) % _NBUF)
            return carry

        lax.fori_loop(1, seq // _NBUF, round_body, 0)

        drain(seq - 1, (seq - 1) % _NBUF)
        for i in range(_NBUF - 1):
            g = seq - 1 - i
            store_wait(g, g % _NBUF)

    return k(table, idx)


def kernel(x, table):
    Bt, S = x.shape
    D = table.shape[1]
    assert Bt % _NW == 0 and D == 64
    bpw = Bt // _NW
    idx = x.reshape(Bt * S).astype(jnp.int32)
    out = _sc_gather(table, idx, bpw, S)
    return (
        out.reshape(S, 8, _NW, 8, bpw)
        .transpose(2, 4, 0, 1, 3)
        .reshape(Bt, S, D)
    )
